# baseline (speedup 1.0000x reference)
import functools

import jax
import jax.numpy as jnp
from jax import lax
from jax.experimental import pallas as pl
from jax.experimental.pallas import tpu as pltpu
from jax.experimental.pallas import tpu_sc as plsc

NUM_PAGES = 8192
PAGE_SIZE = 16
SLOTS = 32
MAX_PAGES_PER_SLOT = 256

_NC, _NS, _L = 2, 16, 16
_NW = _NC * _NS
_PS_CHUNK = NUM_PAGES // _NW

_i32 = jnp.int32


def _body(len_hbm, ps_hbm, pm_hbm, seq_hbm, npu_hbm, cp_hbm, cpp_hbm,
          len_v, row_v, ps_v, seq_v, npu_v, cp_v, cpp_v, sem):
    wid = lax.axis_index("c") * _NS + lax.axis_index("s")

    pltpu.sync_copy(len_hbm, len_v)
    l_lo = len_v[pl.ds(0, _L)]
    l_hi = len_v[pl.ds(_L, _L)]

    iota = lax.iota(_i32, _L)
    idx_lo = iota
    idx_hi = iota + _L

    n_lo = (l_lo + (PAGE_SIZE - 1)) >> 4
    n_hi = (l_hi + (PAGE_SIZE - 1)) >> 4
    dec_lo = ((l_lo > 0) & ((l_lo & (PAGE_SIZE - 1)) == 0)).astype(_i32)
    dec_hi = ((l_hi > 0) & ((l_hi & (PAGE_SIZE - 1)) == 0)).astype(_i32)

    zeros = jnp.zeros((_L,), _i32)
    total_prefill = jnp.sum(n_lo + n_hi)
    total_dec = jnp.sum(dec_lo + dec_hi)
    total_final = total_prefill + total_dec

    n_s = jnp.sum(jnp.where(idx_lo == wid, n_lo, zeros)
                  + jnp.where(idx_hi == wid, n_hi, zeros))
    off_s = 1 + jnp.sum(jnp.where(idx_lo < wid, n_lo, zeros)
                        + jnp.where(idx_hi < wid, n_hi, zeros))
    dec_s = jnp.sum(jnp.where(idx_lo == wid, dec_lo, zeros)
                    + jnp.where(idx_hi == wid, dec_hi, zeros))
    rank_s = jnp.sum(jnp.where(idx_lo < wid, dec_lo, zeros)
                     + jnp.where(idx_hi < wid, dec_hi, zeros))
    pdec_s = total_prefill + 1 + rank_s

    for v in range(MAX_PAGES_PER_SLOT // _L):
        k = iota + v * _L
        val = jnp.where(k < n_s, off_s + k, zeros)
        val = jnp.where((k == n_s) & (dec_s > 0),
                        jnp.broadcast_to(pdec_s, (_L,)), val)
        row_v[pl.ds(v * _L, _L)] = val
    row_cp = pltpu.async_copy(row_v, pm_hbm.at[wid], sem)

    base = wid * _PS_CHUNK
    for v in range(_PS_CHUNK // _L):
        p = base + v * _L + iota
        ps_v[pl.ds(v * _L, _L)] = ((p >= 1) & (p <= total_final)).astype(_i32)
    ps_cp = pltpu.async_copy(ps_v, ps_hbm.at[pl.ds(base, _PS_CHUNK)], sem)

    @pl.when(wid == 0)
    def _():
        sum_lo = jnp.sum(n_lo)
        dsum_lo = jnp.sum(dec_lo)
        off_vec_lo = 1 + jnp.cumsum(n_lo) - n_lo
        off_vec_hi = 1 + sum_lo + jnp.cumsum(n_hi) - n_hi
        rank_lo = jnp.cumsum(dec_lo) - dec_lo
        rank_hi = dsum_lo + jnp.cumsum(dec_hi) - dec_hi
        pdec_lo = total_prefill + 1 + rank_lo
        pdec_hi = total_prefill + 1 + rank_hi

        for half, (l, n, dec, off, pdec) in enumerate(
                ((l_lo, n_lo, dec_lo, off_vec_lo, pdec_lo),
                 (l_hi, n_hi, dec_hi, off_vec_hi, pdec_hi))):
            sl = pl.ds(half * _L, _L)
            seq_v[sl] = l + (l > 0).astype(_i32)
            npu_v[sl] = n + dec
            last = jnp.where(n > 0, off + n - 1, zeros)
            cp_v[sl] = jnp.where(dec > 0, pdec, last)
            cpp_v[sl] = jnp.where(l > 0, l & (PAGE_SIZE - 1), zeros)
        pltpu.async_copy(seq_v, seq_hbm, sem)
        pltpu.async_copy(npu_v, npu_hbm, sem)
        pltpu.async_copy(cp_v, cp_hbm, sem)
        cpp_cp = pltpu.async_copy(cpp_v, cpp_hbm, sem)
        cpp_cp.wait()
        cpp_cp.wait()
        cpp_cp.wait()
        cpp_cp.wait()

    row_cp.wait()
    ps_cp.wait()


@jax.jit
def kernel(true_lengths):
    vec32 = jax.ShapeDtypeStruct((SLOTS,), _i32)
    out_type = (
        jax.ShapeDtypeStruct((NUM_PAGES,), _i32),
        jax.ShapeDtypeStruct((SLOTS, MAX_PAGES_PER_SLOT), _i32),
        vec32, vec32, vec32, vec32,
    )
    run = pl.kernel(
        _body,
        out_type=out_type,
        mesh=plsc.VectorSubcoreMesh(core_axis_name="c", subcore_axis_name="s"),
        compiler_params=pltpu.CompilerParams(
            needs_layout_passes=False,
            skip_device_barrier=True,
            disable_bounds_checks=True,
            disable_semaphore_checks=True,
        ),
        scratch_types=[
            pltpu.VMEM((SLOTS,), _i32),
            pltpu.VMEM((MAX_PAGES_PER_SLOT,), _i32),
            pltpu.VMEM((_PS_CHUNK,), _i32),
            pltpu.VMEM((SLOTS,), _i32),
            pltpu.VMEM((SLOTS,), _i32),
            pltpu.VMEM((SLOTS,), _i32),
            pltpu.VMEM((SLOTS,), _i32),
            pltpu.SemaphoreType.DMA,
        ],
    )
    return run(true_lengths.astype(_i32))


# speedup vs baseline: 1.0023x; 1.0023x over previous
import functools

import jax
import jax.numpy as jnp
from jax import lax
from jax.experimental import pallas as pl
from jax.experimental.pallas import tpu as pltpu
from jax.experimental.pallas import tpu_sc as plsc

NUM_PAGES = 8192
PAGE_SIZE = 16
SLOTS = 32
MAX_PAGES_PER_SLOT = 256

_NC, _NS, _L = 2, 16, 16
_NW = _NC * _NS
_PS_CHUNK = NUM_PAGES // _NW

_i32 = jnp.int32


def _body(len_hbm, ps_hbm, pm_hbm, seq_hbm, npu_hbm, cp_hbm, cpp_hbm,
          len_v, row_v, ps_v, seq_v, npu_v, cp_v, cpp_v, sem):
    wid = lax.axis_index("c") * _NS + lax.axis_index("s")

    pltpu.sync_copy(len_hbm, len_v)
    l_lo = len_v[pl.ds(0, _L)]
    l_hi = len_v[pl.ds(_L, _L)]

    iota = lax.iota(_i32, _L)
    idx_lo = iota
    idx_hi = iota + _L

    n_lo = (l_lo + (PAGE_SIZE - 1)) >> 4
    n_hi = (l_hi + (PAGE_SIZE - 1)) >> 4
    dec_lo = ((l_lo > 0) & ((l_lo & (PAGE_SIZE - 1)) == 0)).astype(_i32)
    dec_hi = ((l_hi > 0) & ((l_hi & (PAGE_SIZE - 1)) == 0)).astype(_i32)

    zeros = jnp.zeros((_L,), _i32)
    total_prefill = jnp.sum(n_lo + n_hi)
    total_dec = jnp.sum(dec_lo + dec_hi)
    total_final = total_prefill + total_dec

    n_s = jnp.sum(jnp.where(idx_lo == wid, n_lo, zeros)
                  + jnp.where(idx_hi == wid, n_hi, zeros))
    off_s = 1 + jnp.sum(jnp.where(idx_lo < wid, n_lo, zeros)
                        + jnp.where(idx_hi < wid, n_hi, zeros))
    dec_s = jnp.sum(jnp.where(idx_lo == wid, dec_lo, zeros)
                    + jnp.where(idx_hi == wid, dec_hi, zeros))
    rank_s = jnp.sum(jnp.where(idx_lo < wid, dec_lo, zeros)
                     + jnp.where(idx_hi < wid, dec_hi, zeros))
    pdec_s = total_prefill + 1 + rank_s

    for v in range(MAX_PAGES_PER_SLOT // _L):
        k = iota + v * _L
        val = jnp.where(k < n_s, off_s + k, zeros)
        val = jnp.where((k == n_s) & (dec_s > 0),
                        jnp.broadcast_to(pdec_s, (_L,)), val)
        row_v[pl.ds(v * _L, _L)] = val
    row_cp = pltpu.async_copy(row_v, pm_hbm.at[wid], sem)

    base = wid * _PS_CHUNK
    for v in range(_PS_CHUNK // _L):
        p = base + v * _L + iota
        ps_v[pl.ds(v * _L, _L)] = ((p >= 1) & (p <= total_final)).astype(_i32)
    ps_cp = pltpu.async_copy(ps_v, ps_hbm.at[pl.ds(base, _PS_CHUNK)], sem)

    @pl.when(wid == 0)
    def _():
        for half, l in enumerate((l_lo, l_hi)):
            seq_v[pl.ds(half * _L, _L)] = l + (l > 0).astype(_i32)
        pltpu.async_copy(seq_v, seq_hbm, sem).wait()

    @pl.when(wid == 1)
    def _():
        for half, (n, dec) in enumerate(((n_lo, dec_lo), (n_hi, dec_hi))):
            npu_v[pl.ds(half * _L, _L)] = n + dec
        pltpu.async_copy(npu_v, npu_hbm, sem).wait()

    @pl.when(wid == 2)
    def _():
        sum_lo = jnp.sum(n_lo)
        dsum_lo = jnp.sum(dec_lo)
        off_vec_lo = 1 + jnp.cumsum(n_lo) - n_lo
        off_vec_hi = 1 + sum_lo + jnp.cumsum(n_hi) - n_hi
        rank_lo = jnp.cumsum(dec_lo) - dec_lo
        rank_hi = dsum_lo + jnp.cumsum(dec_hi) - dec_hi
        for half, (n, dec, off, rank) in enumerate(
                ((n_lo, dec_lo, off_vec_lo, rank_lo),
                 (n_hi, dec_hi, off_vec_hi, rank_hi))):
            last = jnp.where(n > 0, off + n - 1, zeros)
            pdec = total_prefill + 1 + rank
            cp_v[pl.ds(half * _L, _L)] = jnp.where(dec > 0, pdec, last)
        pltpu.async_copy(cp_v, cp_hbm, sem).wait()

    @pl.when(wid == 3)
    def _():
        for half, l in enumerate((l_lo, l_hi)):
            cpp_v[pl.ds(half * _L, _L)] = jnp.where(
                l > 0, l & (PAGE_SIZE - 1), zeros)
        pltpu.async_copy(cpp_v, cpp_hbm, sem).wait()

    row_cp.wait()
    ps_cp.wait()


@jax.jit
def kernel(true_lengths):
    vec32 = jax.ShapeDtypeStruct((SLOTS,), _i32)
    out_type = (
        jax.ShapeDtypeStruct((NUM_PAGES,), _i32),
        jax.ShapeDtypeStruct((SLOTS, MAX_PAGES_PER_SLOT), _i32),
        vec32, vec32, vec32, vec32,
    )
    run = pl.kernel(
        _body,
        out_type=out_type,
        mesh=plsc.VectorSubcoreMesh(core_axis_name="c", subcore_axis_name="s"),
        compiler_params=pltpu.CompilerParams(
            needs_layout_passes=False,
            skip_device_barrier=True,
            disable_bounds_checks=True,
            disable_semaphore_checks=True,
        ),
        scratch_types=[
            pltpu.VMEM((SLOTS,), _i32),
            pltpu.VMEM((MAX_PAGES_PER_SLOT,), _i32),
            pltpu.VMEM((_PS_CHUNK,), _i32),
            pltpu.VMEM((SLOTS,), _i32),
            pltpu.VMEM((SLOTS,), _i32),
            pltpu.VMEM((SLOTS,), _i32),
            pltpu.VMEM((SLOTS,), _i32),
            pltpu.SemaphoreType.DMA,
        ],
    )
    return run(true_lengths.astype(_i32))
